# baseline (speedup 1.0000x reference)
import jax
import jax.numpy as jnp
from jax.experimental import pallas as pl
from jax.experimental.pallas import tpu as pltpu

_TEMP = 0.05
_TILE = 2000


def _hm_kernel(x_ref, lab_ref, f_ref, loss_ref, acc_ref, pick_ref):
    t = pl.program_id(0)
    nt = pl.num_programs(0)

    x = x_ref[...]
    nrm = jnp.sqrt(jnp.sum(x * x, axis=1, keepdims=True))
    x = (x / (jnp.maximum(nrm, 1e-12) * _TEMP)).astype(jnp.bfloat16)

    f = f_ref[...]
    logits = jax.lax.dot_general(
        x, f, (((1,), (1,)), ((), ())), preferred_element_type=jnp.float32)
    e = jnp.exp(logits)
    s = jnp.sum(e, axis=1, keepdims=True)

    cols = t * _TILE + jax.lax.broadcasted_iota(jnp.int32, logits.shape, 1)
    hit = cols == lab_ref[...]
    p = jnp.sum(jnp.where(hit, logits, 0.0), axis=1, keepdims=True)

    @pl.when(t == 0)
    def _():
        acc_ref[...] = s
        pick_ref[...] = p

    @pl.when(t != 0)
    def _():
        acc_ref[...] += s
        pick_ref[...] += p

    @pl.when(t == nt - 1)
    def _():
        prob = jnp.exp(pick_ref[...]) / acc_ref[...]
        loss_ref[...] = -jnp.mean(jnp.log(prob + 1e-6),
                                  axis=(0, 1), keepdims=True)


def kernel(inputs, labels, index, average_center, features):
    B, nfeat = inputs.shape
    n = features.shape[0]
    labs = labels.astype(jnp.int32).reshape(B, 1)
    loss = pl.pallas_call(
        _hm_kernel,
        grid=(n // _TILE,),
        in_specs=[
            pl.BlockSpec((B, nfeat), lambda t: (0, 0)),
            pl.BlockSpec((B, 1), lambda t: (0, 0)),
            pl.BlockSpec((_TILE, nfeat), lambda t: (t, 0)),
        ],
        out_specs=pl.BlockSpec((1, 1), lambda t: (0, 0)),
        out_shape=jax.ShapeDtypeStruct((1, 1), jnp.float32),
        scratch_shapes=[
            pltpu.VMEM((B, 1), jnp.float32),
            pltpu.VMEM((B, 1), jnp.float32),
        ],
    )(inputs, labs, features.astype(jnp.bfloat16))
    return loss[0, 0]


# speedup vs baseline: 1.1470x; 1.1470x over previous
import jax
import jax.numpy as jnp
from jax.experimental import pallas as pl
from jax.experimental.pallas import tpu as pltpu

_TEMP = 0.05
_TILE = 2000


def _hm_kernel(x_ref, lab_ref, f_ref, loss_ref, acc_ref, pick_ref):
    t = pl.program_id(0)
    nt = pl.num_programs(0)

    x = x_ref[...]
    nrm = jnp.sqrt(jnp.sum(x * x, axis=1, keepdims=True))
    x = (x / (jnp.maximum(nrm, 1e-12) * _TEMP)).astype(jnp.bfloat16)

    f = f_ref[...].astype(jnp.bfloat16)
    logits = jax.lax.dot_general(
        x, f, (((1,), (1,)), ((), ())), preferred_element_type=jnp.float32)
    e = jnp.exp(logits)
    s = jnp.sum(e, axis=1, keepdims=True)

    cols = t * _TILE + jax.lax.broadcasted_iota(jnp.int32, logits.shape, 1)
    hit = cols == lab_ref[...]
    p = jnp.sum(jnp.where(hit, logits, 0.0), axis=1, keepdims=True)

    @pl.when(t == 0)
    def _():
        acc_ref[...] = s
        pick_ref[...] = p

    @pl.when(t != 0)
    def _():
        acc_ref[...] += s
        pick_ref[...] += p

    @pl.when(t == nt - 1)
    def _():
        prob = jnp.exp(pick_ref[...]) / acc_ref[...]
        loss_ref[...] = -jnp.mean(jnp.log(prob + 1e-6),
                                  axis=(0, 1), keepdims=True)


def kernel(inputs, labels, index, average_center, features):
    B, nfeat = inputs.shape
    n = features.shape[0]
    labs = labels.astype(jnp.int32).reshape(B, 1)
    loss = pl.pallas_call(
        _hm_kernel,
        grid=(n // _TILE,),
        in_specs=[
            pl.BlockSpec((B, nfeat), lambda t: (0, 0)),
            pl.BlockSpec((B, 1), lambda t: (0, 0)),
            pl.BlockSpec((_TILE, nfeat), lambda t: (t, 0)),
        ],
        out_specs=pl.BlockSpec((1, 1), lambda t: (0, 0)),
        out_shape=jax.ShapeDtypeStruct((1, 1), jnp.float32),
        scratch_shapes=[
            pltpu.VMEM((B, 1), jnp.float32),
            pltpu.VMEM((B, 1), jnp.float32),
        ],
    )(inputs, labs, features)
    return loss[0, 0]


# speedup vs baseline: 1.5703x; 1.3691x over previous
import functools

import jax
import jax.numpy as jnp
from jax import lax
from jax.experimental import pallas as pl
from jax.experimental.pallas import tpu as pltpu
from jax.experimental.pallas import tpu_sc as plsc

_TEMP = 0.05
_TILE = 2000


def _sum_kernel(x_ref, f_ref, acc_ref, xn_ref):
    t = pl.program_id(0)

    @pl.when(t == 0)
    def _():
        x = x_ref[...]
        nrm = jnp.sqrt(jnp.sum(x * x, axis=1, keepdims=True))
        xn_ref[...] = (x / (jnp.maximum(nrm, 1e-12) * _TEMP)).astype(
            jnp.bfloat16)

    f = f_ref[...].astype(jnp.bfloat16)
    logits = jax.lax.dot_general(
        xn_ref[...], f, (((1,), (1,)), ((), ())),
        preferred_element_type=jnp.float32)
    s = jnp.sum(jnp.exp(logits), axis=1, keepdims=True)

    @pl.when(t == 0)
    def _():
        acc_ref[...] = s

    @pl.when(t != 0)
    def _():
        acc_ref[...] += s


def _loss_kernel(x_ref, g_ref, acc_ref, loss_ref):
    x = x_ref[...]
    nrm = jnp.sqrt(jnp.sum(x * x, axis=1, keepdims=True))
    x = x / (jnp.maximum(nrm, 1e-12) * _TEMP)
    picked = jnp.sum(x * g_ref[...], axis=1, keepdims=True)
    prob = jnp.exp(picked) / acc_ref[...]
    loss_ref[...] = -jnp.mean(jnp.log(prob + 1e-6), axis=(0, 1),
                              keepdims=True)


def _gather_rows(features, labels):
    v, d = features.shape
    b = labels.shape[0]
    info = plsc.get_sparse_core_info()
    nc, ns = info.num_cores, info.num_subcores
    nw = nc * ns
    bpw = b // nw
    mesh = plsc.VectorSubcoreMesh(core_axis_name="c", subcore_axis_name="s")

    @functools.partial(
        pl.kernel, mesh=mesh,
        out_type=jax.ShapeDtypeStruct((b, d), jnp.float32),
        scratch_types=[
            pltpu.VMEM((bpw,), jnp.int32),
            pltpu.VMEM((bpw, d), jnp.float32),
            pltpu.SemaphoreType.DMA,
        ],
    )
    def gk(table_hbm, idx_hbm, out_hbm, idx_v, rows_v, sem):
        wid = lax.axis_index("s") * nc + lax.axis_index("c")
        base = wid * bpw
        pltpu.sync_copy(idx_hbm.at[pl.ds(base, bpw)], idx_v)
        pltpu.async_copy(table_hbm.at[idx_v], rows_v, sem).wait()
        pltpu.sync_copy(rows_v, out_hbm.at[pl.ds(base, bpw)])

    return gk(features, labels)


def kernel(inputs, labels, index, average_center, features):
    B, nfeat = inputs.shape
    n = features.shape[0]
    labs = labels.astype(jnp.int32)

    g = _gather_rows(features, labs)

    acc = pl.pallas_call(
        _sum_kernel,
        grid=(n // _TILE,),
        in_specs=[
            pl.BlockSpec((B, nfeat), lambda t: (0, 0)),
            pl.BlockSpec((_TILE, nfeat), lambda t: (t, 0)),
        ],
        out_specs=pl.BlockSpec((B, 1), lambda t: (0, 0)),
        out_shape=jax.ShapeDtypeStruct((B, 1), jnp.float32),
        scratch_shapes=[pltpu.VMEM((B, nfeat), jnp.bfloat16)],
    )(inputs, features)

    loss = pl.pallas_call(
        _loss_kernel,
        in_specs=[
            pl.BlockSpec((B, nfeat), lambda: (0, 0)),
            pl.BlockSpec((B, nfeat), lambda: (0, 0)),
            pl.BlockSpec((B, 1), lambda: (0, 0)),
        ],
        out_specs=pl.BlockSpec((1, 1), lambda: (0, 0)),
        out_shape=jax.ShapeDtypeStruct((1, 1), jnp.float32),
    )(inputs, g, acc)
    return loss[0, 0]


# speedup vs baseline: 1.5768x; 1.0041x over previous
import functools

import jax
import jax.numpy as jnp
from jax import lax
from jax.experimental import pallas as pl
from jax.experimental.pallas import tpu as pltpu
from jax.experimental.pallas import tpu_sc as plsc

_TEMP = 0.05
_TILE = 2000


def _sum_kernel(x_ref, f_ref, acc_ref, xn_ref):
    t = pl.program_id(0)

    @pl.when(t == 0)
    def _():
        x = x_ref[...]
        nrm = jnp.sqrt(jnp.sum(x * x, axis=1, keepdims=True))
        scale = 1.4426950408889634 / (jnp.maximum(nrm, 1e-12) * _TEMP)
        xn_ref[...] = (x * scale).astype(jnp.bfloat16)

    f = f_ref[...].astype(jnp.bfloat16)
    logits = jax.lax.dot_general(
        xn_ref[...], f, (((1,), (1,)), ((), ())),
        preferred_element_type=jnp.float32)
    s = jnp.sum(jnp.exp2(logits), axis=1, keepdims=True)

    @pl.when(t == 0)
    def _():
        acc_ref[...] = s

    @pl.when(t != 0)
    def _():
        acc_ref[...] += s


def _loss_kernel(x_ref, g_ref, acc_ref, loss_ref):
    x = x_ref[...]
    nrm = jnp.sqrt(jnp.sum(x * x, axis=1, keepdims=True))
    x = x / (jnp.maximum(nrm, 1e-12) * _TEMP)
    picked = jnp.sum(x * g_ref[...], axis=1, keepdims=True)
    prob = jnp.exp(picked) / acc_ref[...]
    loss_ref[...] = -jnp.mean(jnp.log(prob + 1e-6), axis=(0, 1),
                              keepdims=True)


def _gather_rows(features, labels):
    v, d = features.shape
    b = labels.shape[0]
    info = plsc.get_sparse_core_info()
    nc, ns = info.num_cores, info.num_subcores
    nw = nc * ns
    bpw = b // nw
    mesh = plsc.VectorSubcoreMesh(core_axis_name="c", subcore_axis_name="s")

    @functools.partial(
        pl.kernel, mesh=mesh,
        out_type=jax.ShapeDtypeStruct((b, d), jnp.float32),
        scratch_types=[
            pltpu.VMEM((bpw,), jnp.int32),
            pltpu.VMEM((bpw, d), jnp.float32),
            pltpu.SemaphoreType.DMA,
        ],
    )
    def gk(table_hbm, idx_hbm, out_hbm, idx_v, rows_v, sem):
        wid = lax.axis_index("s") * nc + lax.axis_index("c")
        base = wid * bpw
        pltpu.sync_copy(idx_hbm.at[pl.ds(base, bpw)], idx_v)
        pltpu.async_copy(table_hbm.at[idx_v], rows_v, sem).wait()
        pltpu.sync_copy(rows_v, out_hbm.at[pl.ds(base, bpw)])

    return gk(features, labels)


def kernel(inputs, labels, index, average_center, features):
    B, nfeat = inputs.shape
    n = features.shape[0]
    labs = labels.astype(jnp.int32)

    g = _gather_rows(features, labs)

    acc = pl.pallas_call(
        _sum_kernel,
        grid=(n // _TILE,),
        in_specs=[
            pl.BlockSpec((B, nfeat), lambda t: (0, 0)),
            pl.BlockSpec((_TILE, nfeat), lambda t: (t, 0)),
        ],
        out_specs=pl.BlockSpec((B, 1), lambda t: (0, 0)),
        out_shape=jax.ShapeDtypeStruct((B, 1), jnp.float32),
        scratch_shapes=[pltpu.VMEM((B, nfeat), jnp.bfloat16)],
    )(inputs, features)

    loss = pl.pallas_call(
        _loss_kernel,
        in_specs=[
            pl.BlockSpec((B, nfeat), lambda: (0, 0)),
            pl.BlockSpec((B, nfeat), lambda: (0, 0)),
            pl.BlockSpec((B, 1), lambda: (0, 0)),
        ],
        out_specs=pl.BlockSpec((1, 1), lambda: (0, 0)),
        out_shape=jax.ShapeDtypeStruct((1, 1), jnp.float32),
    )(inputs, g, acc)
    return loss[0, 0]


# speedup vs baseline: 1.8884x; 1.1976x over previous
import functools

import jax
import jax.numpy as jnp
from jax import lax
from jax.experimental import pallas as pl
from jax.experimental.pallas import tpu as pltpu
from jax.experimental.pallas import tpu_sc as plsc

_TEMP = 0.05
_TILE = 10000


def _sum_kernel(x_ref, f_ref, acc_ref, xn_ref):
    t = pl.program_id(0)

    @pl.when(t == 0)
    def _():
        x = x_ref[...]
        nrm = jnp.sqrt(jnp.sum(x * x, axis=1, keepdims=True))
        scale = 1.4426950408889634 / (jnp.maximum(nrm, 1e-12) * _TEMP)
        xn_ref[...] = (x * scale).astype(jnp.bfloat16)

    f = f_ref[...].astype(jnp.bfloat16)
    logits = jax.lax.dot_general(
        xn_ref[...], f, (((1,), (1,)), ((), ())),
        preferred_element_type=jnp.float32)
    s = jnp.sum(jnp.exp2(logits), axis=1, keepdims=True)

    @pl.when(t == 0)
    def _():
        acc_ref[...] = s

    @pl.when(t != 0)
    def _():
        acc_ref[...] += s


def _loss_kernel(x_ref, g_ref, acc_ref, loss_ref):
    x = x_ref[...]
    nrm = jnp.sqrt(jnp.sum(x * x, axis=1, keepdims=True))
    x = x / (jnp.maximum(nrm, 1e-12) * _TEMP)
    picked = jnp.sum(x * g_ref[...], axis=1, keepdims=True)
    prob = jnp.exp(picked) / acc_ref[...]
    loss_ref[...] = -jnp.mean(jnp.log(prob + 1e-6), axis=(0, 1),
                              keepdims=True)


def _gather_rows(features, labels):
    v, d = features.shape
    b = labels.shape[0]
    info = plsc.get_sparse_core_info()
    nc, ns = info.num_cores, info.num_subcores
    nw = nc * ns
    bpw = b // nw
    mesh = plsc.VectorSubcoreMesh(core_axis_name="c", subcore_axis_name="s")

    @functools.partial(
        pl.kernel, mesh=mesh,
        out_type=jax.ShapeDtypeStruct((b, d), jnp.float32),
        scratch_types=[
            pltpu.VMEM((bpw,), jnp.int32),
            pltpu.VMEM((bpw, d), jnp.float32),
            pltpu.SemaphoreType.DMA,
        ],
    )
    def gk(table_hbm, idx_hbm, out_hbm, idx_v, rows_v, sem):
        wid = lax.axis_index("s") * nc + lax.axis_index("c")
        base = wid * bpw
        pltpu.sync_copy(idx_hbm.at[pl.ds(base, bpw)], idx_v)
        pltpu.async_copy(table_hbm.at[idx_v], rows_v, sem).wait()
        pltpu.sync_copy(rows_v, out_hbm.at[pl.ds(base, bpw)])

    return gk(features, labels)


def kernel(inputs, labels, index, average_center, features):
    B, nfeat = inputs.shape
    n = features.shape[0]
    labs = labels.astype(jnp.int32)

    g = _gather_rows(features, labs)

    acc = pl.pallas_call(
        _sum_kernel,
        grid=(n // _TILE,),
        in_specs=[
            pl.BlockSpec((B, nfeat), lambda t: (0, 0)),
            pl.BlockSpec((_TILE, nfeat), lambda t: (t, 0)),
        ],
        out_specs=pl.BlockSpec((B, 1), lambda t: (0, 0)),
        out_shape=jax.ShapeDtypeStruct((B, 1), jnp.float32),
        scratch_shapes=[pltpu.VMEM((B, nfeat), jnp.bfloat16)],
    )(inputs, features)

    loss = pl.pallas_call(
        _loss_kernel,
        in_specs=[
            pl.BlockSpec((B, nfeat), lambda: (0, 0)),
            pl.BlockSpec((B, nfeat), lambda: (0, 0)),
            pl.BlockSpec((B, 1), lambda: (0, 0)),
        ],
        out_specs=pl.BlockSpec((1, 1), lambda: (0, 0)),
        out_shape=jax.ShapeDtypeStruct((1, 1), jnp.float32),
    )(inputs, g, acc)
    return loss[0, 0]


# speedup vs baseline: 1.9117x; 1.0123x over previous
import functools

import jax
import jax.numpy as jnp
from jax import lax
from jax.experimental import pallas as pl
from jax.experimental.pallas import tpu as pltpu
from jax.experimental.pallas import tpu_sc as plsc

_TEMP = 0.05
_TILE = 20000


def _sum_kernel(x_ref, f_ref, acc_ref, xn_ref):
    t = pl.program_id(0)

    @pl.when(t == 0)
    def _():
        x = x_ref[...]
        nrm = jnp.sqrt(jnp.sum(x * x, axis=1, keepdims=True))
        scale = 1.4426950408889634 / (jnp.maximum(nrm, 1e-12) * _TEMP)
        xn_ref[...] = (x * scale).astype(jnp.bfloat16)

    f = f_ref[...].astype(jnp.bfloat16)
    logits = jax.lax.dot_general(
        xn_ref[...], f, (((1,), (1,)), ((), ())),
        preferred_element_type=jnp.float32)
    s = jnp.sum(jnp.exp2(logits), axis=1, keepdims=True)

    @pl.when(t == 0)
    def _():
        acc_ref[...] = s

    @pl.when(t != 0)
    def _():
        acc_ref[...] += s


def _loss_kernel(x_ref, g_ref, acc_ref, loss_ref):
    x = x_ref[...]
    nrm = jnp.sqrt(jnp.sum(x * x, axis=1, keepdims=True))
    x = x / (jnp.maximum(nrm, 1e-12) * _TEMP)
    picked = jnp.sum(x * g_ref[...], axis=1, keepdims=True)
    prob = jnp.exp(picked) / acc_ref[...]
    loss_ref[...] = -jnp.mean(jnp.log(prob + 1e-6), axis=(0, 1),
                              keepdims=True)


def _gather_rows(features, labels):
    v, d = features.shape
    b = labels.shape[0]
    info = plsc.get_sparse_core_info()
    nc, ns = info.num_cores, info.num_subcores
    nw = nc * ns
    bpw = b // nw
    mesh = plsc.VectorSubcoreMesh(core_axis_name="c", subcore_axis_name="s")

    @functools.partial(
        pl.kernel, mesh=mesh,
        out_type=jax.ShapeDtypeStruct((b, d), jnp.float32),
        scratch_types=[
            pltpu.VMEM((bpw,), jnp.int32),
            pltpu.VMEM((bpw, d), jnp.float32),
            pltpu.SemaphoreType.DMA,
        ],
    )
    def gk(table_hbm, idx_hbm, out_hbm, idx_v, rows_v, sem):
        wid = lax.axis_index("s") * nc + lax.axis_index("c")
        base = wid * bpw
        pltpu.sync_copy(idx_hbm.at[pl.ds(base, bpw)], idx_v)
        pltpu.async_copy(table_hbm.at[idx_v], rows_v, sem).wait()
        pltpu.sync_copy(rows_v, out_hbm.at[pl.ds(base, bpw)])

    return gk(features, labels)


def kernel(inputs, labels, index, average_center, features):
    B, nfeat = inputs.shape
    n = features.shape[0]
    labs = labels.astype(jnp.int32)

    g = _gather_rows(features, labs)

    acc = pl.pallas_call(
        _sum_kernel,
        grid=(n // _TILE,),
        in_specs=[
            pl.BlockSpec((B, nfeat), lambda t: (0, 0)),
            pl.BlockSpec((_TILE, nfeat), lambda t: (t, 0)),
        ],
        out_specs=pl.BlockSpec((B, 1), lambda t: (0, 0)),
        out_shape=jax.ShapeDtypeStruct((B, 1), jnp.float32),
        scratch_shapes=[pltpu.VMEM((B, nfeat), jnp.bfloat16)],
    )(inputs, features)

    loss = pl.pallas_call(
        _loss_kernel,
        in_specs=[
            pl.BlockSpec((B, nfeat), lambda: (0, 0)),
            pl.BlockSpec((B, nfeat), lambda: (0, 0)),
            pl.BlockSpec((B, 1), lambda: (0, 0)),
        ],
        out_specs=pl.BlockSpec((1, 1), lambda: (0, 0)),
        out_shape=jax.ShapeDtypeStruct((1, 1), jnp.float32),
    )(inputs, g, acc)
    return loss[0, 0]
